# Initial kernel scaffold; baseline (speedup 1.0000x reference)
import jax
import jax.numpy as jnp
from jax.experimental import pallas as pl


def kernel(x, u, W1, a_src1, a_dst1, b1, W2, a_src2, a_dst2, b2, W3, a_src3, a_dst3, b3, g1, be1, g2, be2, g3, be3, Wc1, bc1, Wc2, bc2, Wc3, bc3, edge_index, batch):
    raise NotImplementedError("write your pallas kernel here")


# speedup vs baseline: 1.1233x; 1.1233x over previous
import jax
import jax.numpy as jnp
from jax.experimental import pallas as pl
from jax.experimental.pallas import tpu as pltpu

N = 50000
E = 800000
G = 64
F_IN = 128
HID = 16
HEADS = 4
N_GLOBAL = 10


def _gat_conv(x, src, dst, W, a_src, a_dst, b, heads, out_ch, concat):
    n = x.shape[0]
    h = (x @ W).reshape(n, heads, out_ch)
    alpha_src = (h * a_src).sum(-1)
    alpha_dst = (h * a_dst).sum(-1)
    e = alpha_src[src] + alpha_dst[dst]
    e = jax.nn.leaky_relu(e, 0.2)
    ex = jnp.exp(e)
    denom = jax.ops.segment_sum(ex, dst, num_segments=n)
    msg = h[src] * ex[:, :, None]
    out = jax.ops.segment_sum(msg, dst, num_segments=n)
    out = out / (denom + 1e-16)[:, :, None]
    if concat:
        out = out.reshape(n, heads * out_ch)
    else:
        out = out.mean(axis=1)
    return out + b


def _bn_eval(x, g, b):
    inv = 1.0 / jnp.sqrt(1.0 + 1e-5)
    return x * inv * g + b


def _mlp_kernel(z_ref, w1_ref, b1_ref, w2_ref, b2_ref, w3_ref, b3_ref, o_ref):
    z = z_ref[...]
    z = jnp.maximum(z @ w1_ref[...] + b1_ref[...], 0.0)
    z = jnp.maximum(z @ w2_ref[...] + b2_ref[...], 0.0)
    z = jax.nn.sigmoid(z @ w3_ref[...] + b3_ref[...])
    o_ref[...] = z


def kernel(x, u, W1, a_src1, a_dst1, b1, W2, a_src2, a_dst2, b2, W3, a_src3, a_dst3, b3, g1, be1, g2, be2, g3, be3, Wc1, bc1, Wc2, bc2, Wc3, bc3, edge_index, batch):
    n = x.shape[0]
    loops = jnp.arange(n, dtype=edge_index.dtype)
    src = jnp.concatenate([edge_index[0], loops])
    dst = jnp.concatenate([edge_index[1], loops])
    h = _gat_conv(x, src, dst, W1, a_src1, a_dst1, b1, HEADS, HID, True)
    h = jax.nn.elu(_bn_eval(h, g1, be1))
    h = _gat_conv(h, src, dst, W2, a_src2, a_dst2, b2, HEADS, HID, True)
    h = jax.nn.elu(_bn_eval(h, g2, be2))
    h = _gat_conv(h, src, dst, W3, a_src3, a_dst3, b3, 1, HID, False)
    h = jax.nn.elu(_bn_eval(h, g3, be3))
    cnt = jax.ops.segment_sum(jnp.ones((n,), jnp.float32), batch, num_segments=G)
    x_mean = jax.ops.segment_sum(h, batch, num_segments=G) / jnp.maximum(cnt, 1.0)[:, None]
    x_max = jax.ops.segment_max(h, batch, num_segments=G)
    x_max = jnp.where(jnp.isfinite(x_max), x_max, 0.0)
    z = jnp.concatenate([x_mean, x_max, u], axis=1)
    out = pl.pallas_call(
        _mlp_kernel,
        out_shape=jax.ShapeDtypeStruct((G, 1), jnp.float32),
    )(z, Wc1, bc1, Wc2, bc2, Wc3, bc3)
    return out[:, 0]


# speedup vs baseline: 45.3166x; 40.3407x over previous
import functools

import jax
import jax.numpy as jnp
from jax import lax
from jax.experimental import pallas as pl
from jax.experimental.pallas import tpu as pltpu
from jax.experimental.pallas import tpu_sc as plsc

N = 50000
E = 800000
G = 64
F_IN = 128
HID = 16
HEADS = 4
N_GLOBAL = 10

EE = E + N
NW = 32
W1E = 2048
W2E = 128
EP = 851968
NPAD = 50048
ROWS_PER_SUB = NPAD // 16
NP4 = 50176
POOL_CHUNK = NP4 // NW
POOL_WIN = 224
GT = 128

_MESH = plsc.VectorSubcoreMesh(core_axis_name="c", subcore_axis_name="s")


def _f32(shape):
    return jax.ShapeDtypeStruct(shape, jnp.float32)


def _make_edge_logits(H):
    span = EP // NW
    n_win = span // W1E
    QG = W1E // 128

    def body(*refs):
        a_hbm = refs[:H]
        b_hbm = refs[H:2 * H]
        src, dst, ex_out = refs[2 * H:2 * H + 3]
        rest = refs[2 * H + 3:]
        atabs = rest[:H]
        btabs = rest[H:2 * H]
        srcw, dstw, ea, eb, exb, tbuf, sem, sem2 = rest[2 * H:]
        c = lax.axis_index("c")
        s = lax.axis_index("s")
        wid = s * 2 + c
        base = wid * span

        for k in range(H):
            @pl.when(s == k)
            def _stage_a(k=k):
                pltpu.sync_copy(a_hbm[k], tbuf)
                pltpu.sync_copy(tbuf, atabs[k])

            @pl.when(s == H + k)
            def _stage_b(k=k):
                pltpu.sync_copy(b_hbm[k], tbuf)
                pltpu.sync_copy(tbuf, btabs[k])

        plsc.subcore_barrier()

        def win(i, _):
            off = pl.multiple_of(base + i * W1E, 128)
            pltpu.sync_copy(src.at[pl.ds(off, W1E)], srcw)
            pltpu.sync_copy(dst.at[pl.ds(off, W1E)], dstw)
            for head in range(H):
                cps = []
                for q in range(QG):
                    cps.append(pltpu.async_copy(
                        atabs[head].at[srcw.at[pl.ds(q * 128, 128)]],
                        ea.at[pl.ds(q * 128, 128)], sem))
                    cps.append(pltpu.async_copy(
                        btabs[head].at[dstw.at[pl.ds(q * 128, 128)]],
                        eb.at[pl.ds(q * 128, 128)], sem2))
                for cp in cps:
                    cp.wait()

                def grp(j, _):
                    v = ea[pl.ds(j * 16, 16)] + eb[pl.ds(j * 16, 16)]
                    v = jnp.where(v >= 0.0, v, v * jnp.float32(0.2))
                    v = jnp.exp(v)
                    eid = lax.iota(jnp.int32, 16) + (off + j * 16)
                    v = jnp.where(eid < EE, v, jnp.float32(0.0))
                    exb[pl.ds(j * 16, 16)] = v
                    return 0

                lax.fori_loop(0, W1E // 16, grp, 0)
                pltpu.sync_copy(
                    exb, ex_out.at[pl.ds(pl.multiple_of(head * EP + off, 128),
                                         W1E)])
            return 0

        lax.fori_loop(0, n_win, win, 0)

    return pl.kernel(
        body,
        out_type=_f32((H * EP,)),
        mesh=_MESH,
        compiler_params=pltpu.CompilerParams(use_tc_tiling_on_sc=False),
        scratch_types=(
            [pltpu.VMEM_SHARED((N,), jnp.float32) for _ in range(2 * H)]
            + [
                pltpu.VMEM((W1E,), jnp.int32),
                pltpu.VMEM((W1E,), jnp.int32),
                pltpu.VMEM((W1E,), jnp.float32),
                pltpu.VMEM((W1E,), jnp.float32),
                pltpu.VMEM((W1E,), jnp.float32),
                pltpu.VMEM((N,), jnp.float32),
                pltpu.SemaphoreType.DMA,
                pltpu.SemaphoreType.DMA,
            ]
        ),
    )


def _aggregate_body(hcat, ex, src, dst, acc_out, den_out,
                    acc_s, den_s, srcw, dstw, srca, didx0, didx1, ex0w, ex1w,
                    hrows, msg, vb, vbd, sem, sem2, sem3):
    c = lax.axis_index("c")
    s = lax.axis_index("s")
    cN = c * N
    h0 = 2 * c
    h1 = 2 * c + 1
    r0 = s * ROWS_PER_SUB

    def zf(i, _):
        vb[i, pl.ds(0, 16)] = jnp.zeros((16,), jnp.float32)
        vb[i, pl.ds(16, 16)] = jnp.zeros((16,), jnp.float32)
        return 0

    lax.fori_loop(0, 184, zf, 0)

    def zfd(i, _):
        vbd[pl.ds(i * 16, 16)] = jnp.zeros((16,), jnp.float32)
        return 0

    lax.fori_loop(0, 23, zfd, 0)
    for k in range(17):
        pltpu.sync_copy(vb, acc_s.at[pl.ds(r0 + k * 184, 184)])
        pltpu.sync_copy(vbd, den_s.at[pl.ds(r0 * 2 + k * 368, 368)])
    plsc.subcore_barrier()

    span = EP // 16
    base = s * span
    n_win = span // W2E

    def win(i, _):
        off = pl.multiple_of(base + i * W2E, 128)
        pltpu.sync_copy(src.at[pl.ds(off, W2E)], srcw)
        pltpu.sync_copy(dst.at[pl.ds(off, W2E)], dstw)
        pltpu.sync_copy(ex.at[pl.ds(pl.multiple_of(h0 * EP + off, 128),
                                     W2E)], ex0w)
        pltpu.sync_copy(ex.at[pl.ds(pl.multiple_of(h1 * EP + off, 128),
                                     W2E)], ex1w)
        for j in range(W2E // 16):
            sl = pl.ds(j * 16, 16)
            sv = srcw[sl]
            dv = dstw[sl]
            srca[sl] = sv + cN
            didx0[sl] = dv * 2
            didx1[sl] = dv * 2 + 1
        pltpu.async_copy(hcat.at[srca], hrows, sem).wait()

        def edge(i2, _):
            e0 = i2 * 16
            x0v = ex0w[pl.ds(e0, 16)]
            x1v = ex1w[pl.ds(e0, 16)]
            for j in range(16):
                msg[e0 + j, pl.ds(0, 16)] = hrows[e0 + j, pl.ds(0, 16)] * x0v[j]
                msg[e0 + j, pl.ds(16, 16)] = hrows[e0 + j, pl.ds(16, 16)] * x1v[j]
            return 0

        lax.fori_loop(0, W2E // 16, edge, 0)
        cps = [pltpu.async_copy(msg, acc_s.at[dstw], sem2, add=True),
               pltpu.async_copy(ex0w, den_s.at[didx0], sem3, add=True),
               pltpu.async_copy(ex1w, den_s.at[didx1], sem3, add=True)]
        for cp in cps:
            cp.wait()
        return 0

    lax.fori_loop(0, n_win, win, 0)
    plsc.subcore_barrier()
    for k in range(17):
        pltpu.sync_copy(acc_s.at[pl.ds(r0 + k * 184, 184)], vb)
        pltpu.sync_copy(vb, acc_out.at[c, pl.ds(r0 + k * 184, 184)])
        pltpu.sync_copy(den_s.at[pl.ds(r0 * 2 + k * 368, 368)], vbd)
        pltpu.sync_copy(vbd, den_out.at[pl.ds(
            pl.multiple_of(c * (NPAD * 2) + r0 * 2 + k * 368, 8), 368)])


_sc_aggregate = pl.kernel(
    _aggregate_body,
    out_type=(_f32((2, NPAD, 32)), _f32((2 * NPAD * 2,))),
    mesh=_MESH,
    compiler_params=pltpu.CompilerParams(use_tc_tiling_on_sc=False),
    scratch_types=[
        pltpu.VMEM_SHARED((NPAD, 32), jnp.float32),
        pltpu.VMEM_SHARED((NPAD * 2,), jnp.float32),
        pltpu.VMEM((W2E,), jnp.int32),
        pltpu.VMEM((W2E,), jnp.int32),
        pltpu.VMEM((W2E,), jnp.int32),
        pltpu.VMEM((W2E,), jnp.int32),
        pltpu.VMEM((W2E,), jnp.int32),
        pltpu.VMEM((W2E,), jnp.float32),
        pltpu.VMEM((W2E,), jnp.float32),
        pltpu.VMEM((W2E, 32), jnp.float32),
        pltpu.VMEM((W2E, 32), jnp.float32),
        pltpu.VMEM((184, 32), jnp.float32),
        pltpu.VMEM((368,), jnp.float32),
        pltpu.SemaphoreType.DMA,
        pltpu.SemaphoreType.DMA,
        pltpu.SemaphoreType.DMA,
    ],
)


def _aggregate3_body(h3, ex, src, dst, acc_out, den_out,
                     acc_s, den_s, srcw, dstw, ex0w, hrows, msg, vb, vbd,
                     sem, sem2, sem3):
    c = lax.axis_index("c")
    s = lax.axis_index("s")
    wid = s * 2 + c
    r0 = s * ROWS_PER_SUB

    def zf(i, _):
        vb[i, pl.ds(0, 16)] = jnp.zeros((16,), jnp.float32)
        return 0

    lax.fori_loop(0, 184, zf, 0)

    def zfd(i, _):
        vbd[pl.ds(i * 16, 16)] = jnp.zeros((16,), jnp.float32)
        return 0

    lax.fori_loop(0, 23, zfd, 0)
    for k in range(17):
        pltpu.sync_copy(vb, acc_s.at[pl.ds(r0 + k * 184, 184)])
        pltpu.sync_copy(vbd.at[pl.ds(0, 184)],
                        den_s.at[pl.ds(r0 + k * 184, 184)])
    plsc.subcore_barrier()

    span = EP // NW
    base = wid * span
    n_win = span // W2E

    def win(i, _):
        off = pl.multiple_of(base + i * W2E, 128)
        pltpu.sync_copy(src.at[pl.ds(off, W2E)], srcw)
        pltpu.sync_copy(dst.at[pl.ds(off, W2E)], dstw)
        pltpu.sync_copy(ex.at[pl.ds(off, W2E)], ex0w)
        pltpu.async_copy(h3.at[srcw], hrows, sem).wait()

        def edge(i2, _):
            e0 = i2 * 16
            x0v = ex0w[pl.ds(e0, 16)]
            for j in range(16):
                msg[e0 + j, pl.ds(0, 16)] = hrows[e0 + j, pl.ds(0, 16)] * x0v[j]
            return 0

        lax.fori_loop(0, W2E // 16, edge, 0)
        cps = [pltpu.async_copy(msg, acc_s.at[dstw], sem2, add=True),
               pltpu.async_copy(ex0w, den_s.at[dstw], sem3, add=True)]
        for cp in cps:
            cp.wait()
        return 0

    lax.fori_loop(0, n_win, win, 0)
    plsc.subcore_barrier()
    for k in range(17):
        pltpu.sync_copy(acc_s.at[pl.ds(r0 + k * 184, 184)], vb)
        pltpu.sync_copy(vb, acc_out.at[c, pl.ds(r0 + k * 184, 184)])
        pltpu.sync_copy(den_s.at[pl.ds(r0 + k * 184, 184)],
                        vbd.at[pl.ds(0, 184)])
        pltpu.sync_copy(vbd.at[pl.ds(0, 184)], den_out.at[pl.ds(
            pl.multiple_of(c * NPAD + r0 + k * 184, 8), 184)])


_sc_aggregate3 = pl.kernel(
    _aggregate3_body,
    out_type=(_f32((2, NPAD, 16)), _f32((2 * NPAD,))),
    mesh=_MESH,
    compiler_params=pltpu.CompilerParams(use_tc_tiling_on_sc=False),
    scratch_types=[
        pltpu.VMEM_SHARED((NPAD, 16), jnp.float32),
        pltpu.VMEM_SHARED((NPAD,), jnp.float32),
        pltpu.VMEM((W2E,), jnp.int32),
        pltpu.VMEM((W2E,), jnp.int32),
        pltpu.VMEM((W2E,), jnp.float32),
        pltpu.VMEM((W2E, 16), jnp.float32),
        pltpu.VMEM((W2E, 16), jnp.float32),
        pltpu.VMEM((184, 16), jnp.float32),
        pltpu.VMEM((368,), jnp.float32),
        pltpu.SemaphoreType.DMA,
        pltpu.SemaphoreType.DMA,
        pltpu.SemaphoreType.DMA,
    ],
)


def _pool_body(h, batch, maxo, sumo, cnto,
               maxt, sumt, cntt, spmax, spsum, spcnt, hwin, bwin, red, res):
    c = lax.axis_index("c")
    s = lax.axis_index("s")
    wid = s * 2 + c

    def init(r, _):
        maxt[r, pl.ds(0, 16)] = jnp.full((16,), -3e38, jnp.float32)
        sumt[r, pl.ds(0, 16)] = jnp.zeros((16,), jnp.float32)
        cntt[r, pl.ds(0, 16)] = jnp.zeros((16,), jnp.float32)
        return 0

    lax.fori_loop(0, GT, init, 0)

    base = wid * POOL_CHUNK

    def win(i, _):
        off = base + i * POOL_WIN
        pltpu.sync_copy(h.at[pl.ds(off, POOL_WIN)], hwin)
        pltpu.sync_copy(batch.at[pl.ds(off, POOL_WIN)], bwin)

        def row(i, _):
            r0 = i * 16
            bv = bwin[pl.ds(r0, 16)]
            for j in range(16):
                b = bv[j]
                hv = hwin[r0 + j, pl.ds(0, 16)]
                maxt[b, pl.ds(0, 16)] = jnp.maximum(maxt[b, pl.ds(0, 16)], hv)
                sumt[b, pl.ds(0, 16)] = sumt[b, pl.ds(0, 16)] + hv
                cntt[b, pl.ds(0, 16)] = cntt[b, pl.ds(0, 16)] + jnp.float32(1.0)
            return 0

        lax.fori_loop(0, POOL_WIN // 16, row, 0)
        return 0

    lax.fori_loop(0, POOL_CHUNK // POOL_WIN, win, 0)

    pltpu.sync_copy(maxt, spmax.at[s])
    pltpu.sync_copy(sumt, spsum.at[s])
    pltpu.sync_copy(cntt, spcnt.at[s])
    plsc.subcore_barrier()

    rr = GT // 16
    for tab, out in ((spmax, maxo), (spsum, sumo), (spcnt, cnto)):
        pltpu.sync_copy(tab.at[:, pl.ds(s * rr, rr)], red)
        is_max = tab is spmax
        for r5 in range(rr):
            m = red[0, r5, pl.ds(0, 16)]
            for t in range(1, 16):
                v = red[t, r5, pl.ds(0, 16)]
                m = jnp.maximum(m, v) if is_max else m + v
            res[r5, pl.ds(0, 16)] = m
        pltpu.sync_copy(res, out.at[c, pl.ds(s * rr, rr)])


_sc_pool = pl.kernel(
    _pool_body,
    out_type=(_f32((2, GT, 16)), _f32((2, GT, 16)), _f32((2, GT, 16))),
    mesh=_MESH,
    compiler_params=pltpu.CompilerParams(use_tc_tiling_on_sc=False),
    scratch_types=[
        pltpu.VMEM((GT, 16), jnp.float32),
        pltpu.VMEM((GT, 16), jnp.float32),
        pltpu.VMEM((GT, 16), jnp.float32),
        pltpu.VMEM_SHARED((16, GT, 16), jnp.float32),
        pltpu.VMEM_SHARED((16, GT, 16), jnp.float32),
        pltpu.VMEM_SHARED((16, GT, 16), jnp.float32),
        pltpu.VMEM((POOL_WIN, 16), jnp.float32),
        pltpu.VMEM((POOL_WIN,), jnp.int32),
        pltpu.VMEM((16, GT // 16, 16), jnp.float32),
        pltpu.VMEM((GT // 16, 16), jnp.float32),
    ],
)


_BR = 1000


def _proj_kernel(x_ref, w_ref, as_ref, ad_ref, h_ref, s_ref, d_ref):
    h = jnp.dot(x_ref[...], w_ref[...], preferred_element_type=jnp.float32)
    h_ref[...] = h
    s_ref[...] = jnp.dot(h, as_ref[...], preferred_element_type=jnp.float32)
    d_ref[...] = jnp.dot(h, ad_ref[...], preferred_element_type=jnp.float32)


def _tc_project(x, Wm, As, Ad):
    k = x.shape[1]
    co = Wm.shape[1]
    ch = As.shape[1]
    return pl.pallas_call(
        _proj_kernel,
        grid=(N // _BR,),
        in_specs=[
            pl.BlockSpec((_BR, k), lambda i: (i, 0)),
            pl.BlockSpec((k, co), lambda i: (0, 0)),
            pl.BlockSpec((co, ch), lambda i: (0, 0)),
            pl.BlockSpec((co, ch), lambda i: (0, 0)),
        ],
        out_specs=[
            pl.BlockSpec((_BR, co), lambda i: (i, 0)),
            pl.BlockSpec((_BR, ch), lambda i: (i, 0)),
            pl.BlockSpec((_BR, ch), lambda i: (i, 0)),
        ],
        out_shape=[_f32((N, co)), _f32((N, ch)), _f32((N, ch))],
    )(x, Wm, As, Ad)


import math

_BN_INV = float(1.0 / math.sqrt(1.0 + 1e-5))


def _elu(m):
    neg = jnp.where(m > 0.0, 0.0, m)
    return jnp.where(m > 0.0, m, jnp.exp(neg) - 1.0)


def _comb_kernel(aA_ref, aB_ref, d0_ref, d1_ref, r_ref, b_ref, g_ref,
                 be_ref, w_ref, as_ref, ad_ref, h_ref, s_ref, d_ref):
    m = jnp.concatenate([aA_ref[...], aB_ref[...]], axis=1)
    den4 = jnp.concatenate([d0_ref[...], d1_ref[...]], axis=1)
    denr = jnp.dot(den4, r_ref[...], preferred_element_type=jnp.float32)
    m = m / (denr + 1e-16)
    m = m + b_ref[...]
    m = m * _BN_INV * g_ref[...] + be_ref[...]
    m = _elu(m)
    h = jnp.dot(m, w_ref[...], preferred_element_type=jnp.float32)
    h_ref[...] = h
    s_ref[...] = jnp.dot(h, as_ref[...], preferred_element_type=jnp.float32)
    d_ref[...] = jnp.dot(h, ad_ref[...], preferred_element_type=jnp.float32)


def _tc_combine_project(accA, accB, d0, d1, R, b, g, be, Wm, As, Ad):
    co = Wm.shape[1]
    ch = As.shape[1]
    return pl.pallas_call(
        _comb_kernel,
        grid=(N // _BR,),
        in_specs=[
            pl.BlockSpec((_BR, 32), lambda i: (i, 0)),
            pl.BlockSpec((_BR, 32), lambda i: (i, 0)),
            pl.BlockSpec((_BR, 2), lambda i: (i, 0)),
            pl.BlockSpec((_BR, 2), lambda i: (i, 0)),
            pl.BlockSpec((4, 64), lambda i: (0, 0)),
            pl.BlockSpec((1, 64), lambda i: (0, 0)),
            pl.BlockSpec((1, 64), lambda i: (0, 0)),
            pl.BlockSpec((1, 64), lambda i: (0, 0)),
            pl.BlockSpec((64, co), lambda i: (0, 0)),
            pl.BlockSpec((co, ch), lambda i: (0, 0)),
            pl.BlockSpec((co, ch), lambda i: (0, 0)),
        ],
        out_specs=[
            pl.BlockSpec((_BR, co), lambda i: (i, 0)),
            pl.BlockSpec((_BR, ch), lambda i: (i, 0)),
            pl.BlockSpec((_BR, ch), lambda i: (i, 0)),
        ],
        out_shape=[_f32((N, co)), _f32((N, ch)), _f32((N, ch))],
    )(accA, accB, d0, d1, R, b, g, be, Wm, As, Ad)


def _final_feat_kernel(a0_ref, a1_ref, n0_ref, n1_ref, b_ref, g_ref, be_ref,
                       o_ref):
    m = (a0_ref[...] + a1_ref[...]) / (n0_ref[...] + n1_ref[...] + 1e-16)
    m = m + b_ref[...]
    m = m * _BN_INV * g_ref[...] + be_ref[...]
    o_ref[...] = _elu(m)


def _tc_final_feat(a0, a1, n0, n1, b, g, be):
    return pl.pallas_call(
        _final_feat_kernel,
        grid=(N // _BR,),
        in_specs=[
            pl.BlockSpec((_BR, 16), lambda i: (i, 0)),
            pl.BlockSpec((_BR, 16), lambda i: (i, 0)),
            pl.BlockSpec((_BR, 1), lambda i: (i, 0)),
            pl.BlockSpec((_BR, 1), lambda i: (i, 0)),
            pl.BlockSpec((1, 16), lambda i: (0, 0)),
            pl.BlockSpec((1, 16), lambda i: (0, 0)),
            pl.BlockSpec((1, 16), lambda i: (0, 0)),
        ],
        out_specs=[pl.BlockSpec((_BR, 16), lambda i: (i, 0))],
        out_shape=[_f32((N, 16))],
    )(a0, a1, n0, n1, b, g, be)[0]


def _head_kernel(mx0_ref, mx1_ref, sm0_ref, sm1_ref, ct0_ref, ct1_ref, u_ref,
                 w1_ref, b1_ref, w2_ref, b2_ref, w3_ref, b3_ref, o_ref):
    mx = jnp.maximum(mx0_ref[...], mx1_ref[...])[:G]
    sm = (sm0_ref[...] + sm1_ref[...])[:G]
    ct = (ct0_ref[...] + ct1_ref[...])[:G, 0:1]
    mean = sm / jnp.maximum(ct, 1.0)
    xmax = jnp.where(ct > 0.0, mx, 0.0)
    z = jnp.concatenate([mean, xmax, u_ref[...]], axis=1)
    z = jnp.maximum(jnp.dot(z, w1_ref[...], preferred_element_type=jnp.float32)
                    + b1_ref[...], 0.0)
    z = jnp.maximum(jnp.dot(z, w2_ref[...], preferred_element_type=jnp.float32)
                    + b2_ref[...], 0.0)
    z = jnp.dot(z, w3_ref[...], preferred_element_type=jnp.float32) + b3_ref[...]
    o_ref[...] = 1.0 / (1.0 + jnp.exp(-z))


def _tc_head(mx0, mx1, sm0, sm1, ct0, ct1, u, w1, b1, w2, b2, w3, b3):
    return pl.pallas_call(
        _head_kernel,
        out_shape=_f32((G, 1)),
    )(mx0, mx1, sm0, sm1, ct0, ct1, u, w1, b1, w2, b2, w3, b3)


def _attn_mat(a):
    heads, ch = a.shape
    m = jnp.zeros((heads * ch, heads), jnp.float32)
    for h in range(heads):
        m = m.at[h * ch:(h + 1) * ch, h].set(a[h])
    return m


def kernel(x, u, W1, a_src1, a_dst1, b1, W2, a_src2, a_dst2, b2, W3, a_src3,
           a_dst3, b3, g1, be1, g2, be2, g3, be3, Wc1, bc1, Wc2, bc2, Wc3,
           bc3, edge_index, batch):
    loops = jnp.arange(N, dtype=edge_index.dtype)
    pad = (jnp.arange(EP - EE, dtype=edge_index.dtype) * 97) % N
    src = jnp.concatenate([edge_index[0], loops, pad])
    dst = jnp.concatenate([edge_index[1], loops, pad])

    A1s, A1d = _attn_mat(a_src1), _attn_mat(a_dst1)
    A2s, A2d = _attn_mat(a_src2), _attn_mat(a_dst2)
    A3s, A3d = _attn_mat(a_src3), _attn_mat(a_dst3)
    R = jnp.repeat(jnp.eye(4, dtype=jnp.float32), 16, axis=1)

    edge4 = _make_edge_logits(4)
    edge1 = _make_edge_logits(1)

    def heads_of(a):
        return [a[:, k].copy() for k in range(a.shape[1])]

    h, asrc, adst = _tc_project(x, W1, A1s, A1d)
    ex = edge4(*heads_of(asrc), *heads_of(adst), src, dst)
    hcat = jnp.concatenate([h[:, :32], h[:, 32:]], axis=0)
    acc, den = _sc_aggregate(hcat, ex, src, dst)

    den2 = den.reshape(2, NPAD, 2)
    h, asrc, adst = _tc_combine_project(
        acc[0, :N], acc[1, :N], den2[0, :N], den2[1, :N], R,
        b1.reshape(1, -1), g1.reshape(1, -1), be1.reshape(1, -1),
        W2, A2s, A2d)
    ex = edge4(*heads_of(asrc), *heads_of(adst), src, dst)
    hcat = jnp.concatenate([h[:, :32], h[:, 32:]], axis=0)
    acc, den = _sc_aggregate(hcat, ex, src, dst)

    den2 = den.reshape(2, NPAD, 2)
    h3, asrc3, adst3 = _tc_combine_project(
        acc[0, :N], acc[1, :N], den2[0, :N], den2[1, :N], R,
        b2.reshape(1, -1), g2.reshape(1, -1), be2.reshape(1, -1),
        W3, A3s, A3d)
    ex3 = edge1(*heads_of(asrc3), *heads_of(adst3), src, dst)
    acc3, den3 = _sc_aggregate3(h3, ex3, src, dst)

    den3r = den3.reshape(2, NPAD)
    h3f = _tc_final_feat(acc3[0, :N], acc3[1, :N], den3r[0, :N, None],
                         den3r[1, :N, None],
                         b3.reshape(1, -1), g3.reshape(1, -1),
                         be3.reshape(1, -1))

    h3p = jnp.concatenate([h3f, jnp.zeros((NP4 - N, 16), jnp.float32)], axis=0)
    batchp = jnp.concatenate(
        [batch.astype(jnp.int32), jnp.full((NP4 - N,), G, jnp.int32)])
    mx, sm, ct = _sc_pool(h3p, batchp)
    out = _tc_head(mx[0], mx[1], sm[0], sm[1], ct[0], ct[1], u,
                   Wc1, bc1.reshape(1, -1), Wc2, bc2.reshape(1, -1),
                   Wc3, bc3.reshape(1, -1))
    return out[:, 0]


# speedup vs baseline: 98.3668x; 2.1707x over previous
import functools

import jax
import jax.numpy as jnp
from jax import lax
from jax.experimental import pallas as pl
from jax.experimental.pallas import tpu as pltpu
from jax.experimental.pallas import tpu_sc as plsc

N = 50000
E = 800000
G = 64
F_IN = 128
HID = 16
HEADS = 4
N_GLOBAL = 10

EE = E + N
NW = 32
W1E = 2048
W2E = 1024
EP = 851968
NPAD = 50048
ROWS_PER_SUB = NPAD // 16
NP4 = 50176
POOL_CHUNK = NP4 // NW
POOL_WIN = 224
GT = 128

_MESH = plsc.VectorSubcoreMesh(core_axis_name="c", subcore_axis_name="s")


def _f32(shape):
    return jax.ShapeDtypeStruct(shape, jnp.float32)


def _make_edge_logits(H):
    span = EP // NW
    n_win = span // W1E
    QG = W1E // 128

    def body(*refs):
        a_hbm = refs[:H]
        b_hbm = refs[H:2 * H]
        src, dst, ex_out = refs[2 * H:2 * H + 3]
        rest = refs[2 * H + 3:]
        atabs = rest[:H]
        btabs = rest[H:2 * H]
        srcw, dstw, ea, eb, exb, tbuf, sem, sem2 = rest[2 * H:]
        c = lax.axis_index("c")
        s = lax.axis_index("s")
        wid = s * 2 + c
        base = wid * span

        for k in range(H):
            @pl.when(s == k)
            def _stage_a(k=k):
                pltpu.sync_copy(a_hbm[k], tbuf)
                pltpu.sync_copy(tbuf, atabs[k])

            @pl.when(s == H + k)
            def _stage_b(k=k):
                pltpu.sync_copy(b_hbm[k], tbuf)
                pltpu.sync_copy(tbuf, btabs[k])

        plsc.subcore_barrier()

        def win(i, _):
            off = pl.multiple_of(base + i * W1E, 128)
            pltpu.sync_copy(src.at[pl.ds(off, W1E)], srcw)
            pltpu.sync_copy(dst.at[pl.ds(off, W1E)], dstw)
            for head in range(H):
                cps = []
                for q in range(QG):
                    cps.append(pltpu.async_copy(
                        atabs[head].at[srcw.at[pl.ds(q * 128, 128)]],
                        ea.at[pl.ds(q * 128, 128)], sem))
                    cps.append(pltpu.async_copy(
                        btabs[head].at[dstw.at[pl.ds(q * 128, 128)]],
                        eb.at[pl.ds(q * 128, 128)], sem2))
                for cp in cps:
                    cp.wait()

                def grp(j, _):
                    v = ea[pl.ds(j * 16, 16)] + eb[pl.ds(j * 16, 16)]
                    v = jnp.where(v >= 0.0, v, v * jnp.float32(0.2))
                    v = jnp.exp(v)
                    eid = lax.iota(jnp.int32, 16) + (off + j * 16)
                    v = jnp.where(eid < EE, v, jnp.float32(0.0))
                    exb[pl.ds(j * 16, 16)] = v
                    return 0

                lax.fori_loop(0, W1E // 16, grp, 0)
                pltpu.sync_copy(
                    exb, ex_out.at[pl.ds(pl.multiple_of(head * EP + off, 128),
                                         W1E)])
            return 0

        lax.fori_loop(0, n_win, win, 0)

    return pl.kernel(
        body,
        out_type=_f32((H * EP,)),
        mesh=_MESH,
        compiler_params=pltpu.CompilerParams(use_tc_tiling_on_sc=False),
        scratch_types=(
            [pltpu.VMEM_SHARED((N,), jnp.float32) for _ in range(2 * H)]
            + [
                pltpu.VMEM((W1E,), jnp.int32),
                pltpu.VMEM((W1E,), jnp.int32),
                pltpu.VMEM((W1E,), jnp.float32),
                pltpu.VMEM((W1E,), jnp.float32),
                pltpu.VMEM((W1E,), jnp.float32),
                pltpu.VMEM((N,), jnp.float32),
                pltpu.SemaphoreType.DMA,
                pltpu.SemaphoreType.DMA,
            ]
        ),
    )


def _aggregate_body(hcat, ex, src2d, dst2d, acc_out, den_out,
                    acc_s, den_s, srcw, dstw, srca, didx0, didx1, ex0w, ex1w,
                    msg0, msg1, msg2, semg, sems, semd):
    c = lax.axis_index("c")
    s = lax.axis_index("s")
    cN = c * N
    h0 = 2 * c
    h1 = 2 * c + 1
    r0 = s * ROWS_PER_SUB
    msgs = (msg0, msg1, msg2)

    def zf(i, _):
        msg0[i, pl.ds(0, 16)] = jnp.zeros((16,), jnp.float32)
        msg0[i, pl.ds(16, 16)] = jnp.zeros((16,), jnp.float32)
        return 0

    lax.fori_loop(0, 128, zf, 0)

    def zfd(i, _):
        ex0w[pl.ds(i * 16, 16)] = jnp.zeros((16,), jnp.float32)
        return 0

    lax.fori_loop(0, 64, zfd, 0)
    for k in range(24):
        pltpu.sync_copy(msg0, acc_s.at[pl.ds(r0 + k * 128, 128)])
    pltpu.sync_copy(msg0.at[pl.ds(0, 56)], acc_s.at[pl.ds(r0 + 3072, 56)])
    for k in range(6):
        pltpu.sync_copy(ex0w, den_s.at[pl.ds(r0 * 2 + k * 1024, 1024)])
    pltpu.sync_copy(ex0w.at[pl.ds(0, 112)],
                    den_s.at[pl.ds(r0 * 2 + 6144, 112)])
    plsc.subcore_barrier()

    span = EP // 16
    base = s * span
    n_win = span // W2E

    def win(i, _):
        off = pl.multiple_of(base + i * W2E, 128)
        row0 = pl.multiple_of(off // 128, 8)
        pltpu.sync_copy(src2d.at[pl.ds(row0, 8)], srcw)
        pltpu.sync_copy(dst2d.at[pl.ds(row0, 8)], dstw)
        pltpu.sync_copy(ex.at[pl.ds(pl.multiple_of(h0 * EP + off, 128),
                                     W2E)], ex0w)
        pltpu.sync_copy(ex.at[pl.ds(pl.multiple_of(h1 * EP + off, 128),
                                     W2E)], ex1w)
        for q in range(8):
            for j in range(8):
                sl = pl.ds(j * 16, 16)
                srca[q, sl] = srcw[q, sl] + cN
                dv = dstw[q, sl]
                didx0[q, sl] = dv * 2
                didx1[q, sl] = dv * 2 + 1

        gath = {0: pltpu.async_copy(hcat.at[srca.at[0]], msgs[0], semg)}
        scat = {}
        for q in range(8):
            if q >= 2:
                for cp in scat[q - 2]:
                    cp.wait()
            if q + 1 < 8:
                gath[q + 1] = pltpu.async_copy(
                    hcat.at[srca.at[q + 1]], msgs[(q + 1) % 3], semg)
            gath[q].wait()
            buf = msgs[q % 3]
            for g in range(8):
                x0v = ex0w[pl.ds(q * 128 + g * 16, 16)]
                x1v = ex1w[pl.ds(q * 128 + g * 16, 16)]
                for j in range(16):
                    e = g * 16 + j
                    buf[e, pl.ds(0, 16)] = buf[e, pl.ds(0, 16)] * x0v[j]
                    buf[e, pl.ds(16, 16)] = buf[e, pl.ds(16, 16)] * x1v[j]
            scat[q] = [
                pltpu.async_copy(buf, acc_s.at[dstw.at[q]], sems, add=True),
                pltpu.async_copy(ex0w.at[pl.ds(q * 128, 128)],
                                 den_s.at[didx0.at[q]], semd, add=True),
                pltpu.async_copy(ex1w.at[pl.ds(q * 128, 128)],
                                 den_s.at[didx1.at[q]], semd, add=True),
            ]
        for q in (6, 7):
            for cp in scat[q]:
                cp.wait()
        return 0

    lax.fori_loop(0, n_win, win, 0)
    plsc.subcore_barrier()
    for k in range(24):
        pltpu.sync_copy(acc_s.at[pl.ds(r0 + k * 128, 128)], msg0)
        pltpu.sync_copy(msg0, acc_out.at[c, pl.ds(r0 + k * 128, 128)])
    pltpu.sync_copy(acc_s.at[pl.ds(r0 + 3072, 56)], msg0.at[pl.ds(0, 56)])
    pltpu.sync_copy(msg0.at[pl.ds(0, 56)],
                    acc_out.at[c, pl.ds(r0 + 3072, 56)])
    dbase = c * (NPAD * 2) + r0 * 2
    for k in range(6):
        pltpu.sync_copy(den_s.at[pl.ds(r0 * 2 + k * 1024, 1024)], ex0w)
        pltpu.sync_copy(ex0w, den_out.at[pl.ds(
            pl.multiple_of(dbase + k * 1024, 8), 1024)])
    pltpu.sync_copy(den_s.at[pl.ds(r0 * 2 + 6144, 112)],
                    ex0w.at[pl.ds(0, 112)])
    pltpu.sync_copy(ex0w.at[pl.ds(0, 112)],
                    den_out.at[pl.ds(pl.multiple_of(dbase + 6144, 8), 112)])


_sc_aggregate = pl.kernel(
    _aggregate_body,
    out_type=(_f32((2, NPAD, 32)), _f32((2 * NPAD * 2,))),
    mesh=_MESH,
    compiler_params=pltpu.CompilerParams(use_tc_tiling_on_sc=False),
    scratch_types=[
        pltpu.VMEM_SHARED((NPAD, 32), jnp.float32),
        pltpu.VMEM_SHARED((NPAD * 2,), jnp.float32),
        pltpu.VMEM((8, 128), jnp.int32),
        pltpu.VMEM((8, 128), jnp.int32),
        pltpu.VMEM((8, 128), jnp.int32),
        pltpu.VMEM((8, 128), jnp.int32),
        pltpu.VMEM((8, 128), jnp.int32),
        pltpu.VMEM((W2E,), jnp.float32),
        pltpu.VMEM((W2E,), jnp.float32),
        pltpu.VMEM((128, 32), jnp.float32),
        pltpu.VMEM((128, 32), jnp.float32),
        pltpu.VMEM((128, 32), jnp.float32),
        pltpu.SemaphoreType.DMA,
        pltpu.SemaphoreType.DMA,
        pltpu.SemaphoreType.DMA,
    ],
)


def _aggregate3_body(h3, ex, src2d, dst2d, acc_out, den_out,
                     acc_s, den_s, srcw, dstw, ex0w, msg0, msg1, msg2,
                     semg, sems, semd):
    c = lax.axis_index("c")
    s = lax.axis_index("s")
    wid = s * 2 + c
    r0 = s * ROWS_PER_SUB
    msgs = (msg0, msg1, msg2)

    def zf(i, _):
        msg0[i, pl.ds(0, 16)] = jnp.zeros((16,), jnp.float32)
        return 0

    lax.fori_loop(0, 128, zf, 0)

    def zfd(i, _):
        ex0w[pl.ds(i * 16, 16)] = jnp.zeros((16,), jnp.float32)
        return 0

    lax.fori_loop(0, 64, zfd, 0)
    for k in range(24):
        pltpu.sync_copy(msg0, acc_s.at[pl.ds(r0 + k * 128, 128)])
    pltpu.sync_copy(msg0.at[pl.ds(0, 56)], acc_s.at[pl.ds(r0 + 3072, 56)])
    for k in range(3):
        pltpu.sync_copy(ex0w, den_s.at[pl.ds(r0 + k * 1024, 1024)])
    pltpu.sync_copy(ex0w.at[pl.ds(0, 56)], den_s.at[pl.ds(r0 + 3072, 56)])
    plsc.subcore_barrier()

    span = EP // NW
    base = wid * span
    n_win = span // W2E

    def win(i, _):
        off = pl.multiple_of(base + i * W2E, 128)
        row0 = pl.multiple_of(off // 128, 8)
        pltpu.sync_copy(src2d.at[pl.ds(row0, 8)], srcw)
        pltpu.sync_copy(dst2d.at[pl.ds(row0, 8)], dstw)
        pltpu.sync_copy(ex.at[pl.ds(off, W2E)], ex0w)

        gath = {0: pltpu.async_copy(h3.at[srcw.at[0]], msgs[0], semg)}
        scat = {}
        for q in range(8):
            if q >= 2:
                for cp in scat[q - 2]:
                    cp.wait()
            if q + 1 < 8:
                gath[q + 1] = pltpu.async_copy(
                    h3.at[srcw.at[q + 1]], msgs[(q + 1) % 3], semg)
            gath[q].wait()
            buf = msgs[q % 3]
            for g in range(8):
                x0v = ex0w[pl.ds(q * 128 + g * 16, 16)]
                for j in range(16):
                    e = g * 16 + j
                    buf[e, pl.ds(0, 16)] = buf[e, pl.ds(0, 16)] * x0v[j]
            scat[q] = [
                pltpu.async_copy(buf, acc_s.at[dstw.at[q]], sems, add=True),
                pltpu.async_copy(ex0w.at[pl.ds(q * 128, 128)],
                                 den_s.at[dstw.at[q]], semd, add=True),
            ]
        for q in (6, 7):
            for cp in scat[q]:
                cp.wait()
        return 0

    lax.fori_loop(0, n_win, win, 0)
    plsc.subcore_barrier()
    for k in range(24):
        pltpu.sync_copy(acc_s.at[pl.ds(r0 + k * 128, 128)], msg0)
        pltpu.sync_copy(msg0, acc_out.at[c, pl.ds(r0 + k * 128, 128)])
    pltpu.sync_copy(acc_s.at[pl.ds(r0 + 3072, 56)], msg0.at[pl.ds(0, 56)])
    pltpu.sync_copy(msg0.at[pl.ds(0, 56)],
                    acc_out.at[c, pl.ds(r0 + 3072, 56)])
    dbase = c * NPAD + r0
    for k in range(3):
        pltpu.sync_copy(den_s.at[pl.ds(r0 + k * 1024, 1024)], ex0w)
        pltpu.sync_copy(ex0w, den_out.at[pl.ds(
            pl.multiple_of(dbase + k * 1024, 8), 1024)])
    pltpu.sync_copy(den_s.at[pl.ds(r0 + 3072, 56)], ex0w.at[pl.ds(0, 56)])
    pltpu.sync_copy(ex0w.at[pl.ds(0, 56)],
                    den_out.at[pl.ds(pl.multiple_of(dbase + 3072, 8), 56)])


_sc_aggregate3 = pl.kernel(
    _aggregate3_body,
    out_type=(_f32((2, NPAD, 16)), _f32((2 * NPAD,))),
    mesh=_MESH,
    compiler_params=pltpu.CompilerParams(use_tc_tiling_on_sc=False),
    scratch_types=[
        pltpu.VMEM_SHARED((NPAD, 16), jnp.float32),
        pltpu.VMEM_SHARED((NPAD,), jnp.float32),
        pltpu.VMEM((8, 128), jnp.int32),
        pltpu.VMEM((8, 128), jnp.int32),
        pltpu.VMEM((W2E,), jnp.float32),
        pltpu.VMEM((128, 16), jnp.float32),
        pltpu.VMEM((128, 16), jnp.float32),
        pltpu.VMEM((128, 16), jnp.float32),
        pltpu.SemaphoreType.DMA,
        pltpu.SemaphoreType.DMA,
        pltpu.SemaphoreType.DMA,
    ],
)


def _pool_body(h, batch, maxo, sumo, cnto,
               maxt, sumt, cntt, spmax, spsum, spcnt, hwin, bwin, red, res):
    c = lax.axis_index("c")
    s = lax.axis_index("s")
    wid = s * 2 + c

    def init(r, _):
        maxt[r, pl.ds(0, 16)] = jnp.full((16,), -3e38, jnp.float32)
        sumt[r, pl.ds(0, 16)] = jnp.zeros((16,), jnp.float32)
        cntt[r, pl.ds(0, 16)] = jnp.zeros((16,), jnp.float32)
        return 0

    lax.fori_loop(0, GT, init, 0)

    base = wid * POOL_CHUNK

    def win(i, _):
        off = base + i * POOL_WIN
        pltpu.sync_copy(h.at[pl.ds(off, POOL_WIN)], hwin)
        pltpu.sync_copy(batch.at[pl.ds(off, POOL_WIN)], bwin)

        def row(i, _):
            r0 = i * 16
            bv = bwin[pl.ds(r0, 16)]
            for j in range(16):
                b = bv[j]
                hv = hwin[r0 + j, pl.ds(0, 16)]
                maxt[b, pl.ds(0, 16)] = jnp.maximum(maxt[b, pl.ds(0, 16)], hv)
                sumt[b, pl.ds(0, 16)] = sumt[b, pl.ds(0, 16)] + hv
                cntt[b, pl.ds(0, 16)] = cntt[b, pl.ds(0, 16)] + jnp.float32(1.0)
            return 0

        lax.fori_loop(0, POOL_WIN // 16, row, 0)
        return 0

    lax.fori_loop(0, POOL_CHUNK // POOL_WIN, win, 0)

    pltpu.sync_copy(maxt, spmax.at[s])
    pltpu.sync_copy(sumt, spsum.at[s])
    pltpu.sync_copy(cntt, spcnt.at[s])
    plsc.subcore_barrier()

    rr = GT // 16
    for tab, out in ((spmax, maxo), (spsum, sumo), (spcnt, cnto)):
        pltpu.sync_copy(tab.at[:, pl.ds(s * rr, rr)], red)
        is_max = tab is spmax
        for r5 in range(rr):
            m = red[0, r5, pl.ds(0, 16)]
            for t in range(1, 16):
                v = red[t, r5, pl.ds(0, 16)]
                m = jnp.maximum(m, v) if is_max else m + v
            res[r5, pl.ds(0, 16)] = m
        pltpu.sync_copy(res, out.at[c, pl.ds(s * rr, rr)])


_sc_pool = pl.kernel(
    _pool_body,
    out_type=(_f32((2, GT, 16)), _f32((2, GT, 16)), _f32((2, GT, 16))),
    mesh=_MESH,
    compiler_params=pltpu.CompilerParams(use_tc_tiling_on_sc=False),
    scratch_types=[
        pltpu.VMEM((GT, 16), jnp.float32),
        pltpu.VMEM((GT, 16), jnp.float32),
        pltpu.VMEM((GT, 16), jnp.float32),
        pltpu.VMEM_SHARED((16, GT, 16), jnp.float32),
        pltpu.VMEM_SHARED((16, GT, 16), jnp.float32),
        pltpu.VMEM_SHARED((16, GT, 16), jnp.float32),
        pltpu.VMEM((POOL_WIN, 16), jnp.float32),
        pltpu.VMEM((POOL_WIN,), jnp.int32),
        pltpu.VMEM((16, GT // 16, 16), jnp.float32),
        pltpu.VMEM((GT // 16, 16), jnp.float32),
    ],
)


_BR = 1000


def _proj_kernel(x_ref, w_ref, as_ref, ad_ref, h_ref, s_ref, d_ref):
    h = jnp.dot(x_ref[...], w_ref[...], preferred_element_type=jnp.float32)
    h_ref[...] = h
    s_ref[...] = jnp.dot(h, as_ref[...], preferred_element_type=jnp.float32)
    d_ref[...] = jnp.dot(h, ad_ref[...], preferred_element_type=jnp.float32)


def _tc_project(x, Wm, As, Ad):
    k = x.shape[1]
    co = Wm.shape[1]
    ch = As.shape[1]
    return pl.pallas_call(
        _proj_kernel,
        grid=(N // _BR,),
        in_specs=[
            pl.BlockSpec((_BR, k), lambda i: (i, 0)),
            pl.BlockSpec((k, co), lambda i: (0, 0)),
            pl.BlockSpec((co, ch), lambda i: (0, 0)),
            pl.BlockSpec((co, ch), lambda i: (0, 0)),
        ],
        out_specs=[
            pl.BlockSpec((_BR, co), lambda i: (i, 0)),
            pl.BlockSpec((_BR, ch), lambda i: (i, 0)),
            pl.BlockSpec((_BR, ch), lambda i: (i, 0)),
        ],
        out_shape=[_f32((N, co)), _f32((N, ch)), _f32((N, ch))],
    )(x, Wm, As, Ad)


import math

_BN_INV = float(1.0 / math.sqrt(1.0 + 1e-5))


def _elu(m):
    neg = jnp.where(m > 0.0, 0.0, m)
    return jnp.where(m > 0.0, m, jnp.exp(neg) - 1.0)


def _comb_kernel(aA_ref, aB_ref, d0_ref, d1_ref, r_ref, b_ref, g_ref,
                 be_ref, w_ref, as_ref, ad_ref, h_ref, s_ref, d_ref):
    m = jnp.concatenate([aA_ref[...], aB_ref[...]], axis=1)
    den4 = jnp.concatenate([d0_ref[...], d1_ref[...]], axis=1)
    denr = jnp.dot(den4, r_ref[...], preferred_element_type=jnp.float32)
    m = m / (denr + 1e-16)
    m = m + b_ref[...]
    m = m * _BN_INV * g_ref[...] + be_ref[...]
    m = _elu(m)
    h = jnp.dot(m, w_ref[...], preferred_element_type=jnp.float32)
    h_ref[...] = h
    s_ref[...] = jnp.dot(h, as_ref[...], preferred_element_type=jnp.float32)
    d_ref[...] = jnp.dot(h, ad_ref[...], preferred_element_type=jnp.float32)


def _tc_combine_project(accA, accB, d0, d1, R, b, g, be, Wm, As, Ad):
    co = Wm.shape[1]
    ch = As.shape[1]
    return pl.pallas_call(
        _comb_kernel,
        grid=(N // _BR,),
        in_specs=[
            pl.BlockSpec((_BR, 32), lambda i: (i, 0)),
            pl.BlockSpec((_BR, 32), lambda i: (i, 0)),
            pl.BlockSpec((_BR, 2), lambda i: (i, 0)),
            pl.BlockSpec((_BR, 2), lambda i: (i, 0)),
            pl.BlockSpec((4, 64), lambda i: (0, 0)),
            pl.BlockSpec((1, 64), lambda i: (0, 0)),
            pl.BlockSpec((1, 64), lambda i: (0, 0)),
            pl.BlockSpec((1, 64), lambda i: (0, 0)),
            pl.BlockSpec((64, co), lambda i: (0, 0)),
            pl.BlockSpec((co, ch), lambda i: (0, 0)),
            pl.BlockSpec((co, ch), lambda i: (0, 0)),
        ],
        out_specs=[
            pl.BlockSpec((_BR, co), lambda i: (i, 0)),
            pl.BlockSpec((_BR, ch), lambda i: (i, 0)),
            pl.BlockSpec((_BR, ch), lambda i: (i, 0)),
        ],
        out_shape=[_f32((N, co)), _f32((N, ch)), _f32((N, ch))],
    )(accA, accB, d0, d1, R, b, g, be, Wm, As, Ad)


def _final_feat_kernel(a0_ref, a1_ref, n0_ref, n1_ref, b_ref, g_ref, be_ref,
                       o_ref):
    m = (a0_ref[...] + a1_ref[...]) / (n0_ref[...] + n1_ref[...] + 1e-16)
    m = m + b_ref[...]
    m = m * _BN_INV * g_ref[...] + be_ref[...]
    o_ref[...] = _elu(m)


def _tc_final_feat(a0, a1, n0, n1, b, g, be):
    return pl.pallas_call(
        _final_feat_kernel,
        grid=(N // _BR,),
        in_specs=[
            pl.BlockSpec((_BR, 16), lambda i: (i, 0)),
            pl.BlockSpec((_BR, 16), lambda i: (i, 0)),
            pl.BlockSpec((_BR, 1), lambda i: (i, 0)),
            pl.BlockSpec((_BR, 1), lambda i: (i, 0)),
            pl.BlockSpec((1, 16), lambda i: (0, 0)),
            pl.BlockSpec((1, 16), lambda i: (0, 0)),
            pl.BlockSpec((1, 16), lambda i: (0, 0)),
        ],
        out_specs=[pl.BlockSpec((_BR, 16), lambda i: (i, 0))],
        out_shape=[_f32((N, 16))],
    )(a0, a1, n0, n1, b, g, be)[0]


def _head_kernel(mx0_ref, mx1_ref, sm0_ref, sm1_ref, ct0_ref, ct1_ref, u_ref,
                 w1_ref, b1_ref, w2_ref, b2_ref, w3_ref, b3_ref, o_ref):
    mx = jnp.maximum(mx0_ref[...], mx1_ref[...])[:G]
    sm = (sm0_ref[...] + sm1_ref[...])[:G]
    ct = (ct0_ref[...] + ct1_ref[...])[:G, 0:1]
    mean = sm / jnp.maximum(ct, 1.0)
    xmax = jnp.where(ct > 0.0, mx, 0.0)
    z = jnp.concatenate([mean, xmax, u_ref[...]], axis=1)
    z = jnp.maximum(jnp.dot(z, w1_ref[...], preferred_element_type=jnp.float32)
                    + b1_ref[...], 0.0)
    z = jnp.maximum(jnp.dot(z, w2_ref[...], preferred_element_type=jnp.float32)
                    + b2_ref[...], 0.0)
    z = jnp.dot(z, w3_ref[...], preferred_element_type=jnp.float32) + b3_ref[...]
    o_ref[...] = 1.0 / (1.0 + jnp.exp(-z))


def _tc_head(mx0, mx1, sm0, sm1, ct0, ct1, u, w1, b1, w2, b2, w3, b3):
    return pl.pallas_call(
        _head_kernel,
        out_shape=_f32((G, 1)),
    )(mx0, mx1, sm0, sm1, ct0, ct1, u, w1, b1, w2, b2, w3, b3)


def _attn_mat(a):
    heads, ch = a.shape
    m = jnp.zeros((heads * ch, heads), jnp.float32)
    for h in range(heads):
        m = m.at[h * ch:(h + 1) * ch, h].set(a[h])
    return m


def kernel(x, u, W1, a_src1, a_dst1, b1, W2, a_src2, a_dst2, b2, W3, a_src3,
           a_dst3, b3, g1, be1, g2, be2, g3, be3, Wc1, bc1, Wc2, bc2, Wc3,
           bc3, edge_index, batch):
    loops = jnp.arange(N, dtype=edge_index.dtype)
    pad = (jnp.arange(EP - EE, dtype=edge_index.dtype) * 97) % N
    src = jnp.concatenate([edge_index[0], loops, pad])
    dst = jnp.concatenate([edge_index[1], loops, pad])
    src2d = src.reshape(EP // 128, 128)
    dst2d = dst.reshape(EP // 128, 128)

    A1s, A1d = _attn_mat(a_src1), _attn_mat(a_dst1)
    A2s, A2d = _attn_mat(a_src2), _attn_mat(a_dst2)
    A3s, A3d = _attn_mat(a_src3), _attn_mat(a_dst3)
    R = jnp.repeat(jnp.eye(4, dtype=jnp.float32), 16, axis=1)

    edge4 = _make_edge_logits(4)
    edge1 = _make_edge_logits(1)

    def heads_of(a):
        return [a[:, k].copy() for k in range(a.shape[1])]

    h, asrc, adst = _tc_project(x, W1, A1s, A1d)
    ex = edge4(*heads_of(asrc), *heads_of(adst), src, dst)
    hcat = jnp.concatenate([h[:, :32], h[:, 32:]], axis=0)
    acc, den = _sc_aggregate(hcat, ex, src2d, dst2d)

    den2 = den.reshape(2, NPAD, 2)
    h, asrc, adst = _tc_combine_project(
        acc[0, :N], acc[1, :N], den2[0, :N], den2[1, :N], R,
        b1.reshape(1, -1), g1.reshape(1, -1), be1.reshape(1, -1),
        W2, A2s, A2d)
    ex = edge4(*heads_of(asrc), *heads_of(adst), src, dst)
    hcat = jnp.concatenate([h[:, :32], h[:, 32:]], axis=0)
    acc, den = _sc_aggregate(hcat, ex, src2d, dst2d)

    den2 = den.reshape(2, NPAD, 2)
    h3, asrc3, adst3 = _tc_combine_project(
        acc[0, :N], acc[1, :N], den2[0, :N], den2[1, :N], R,
        b2.reshape(1, -1), g2.reshape(1, -1), be2.reshape(1, -1),
        W3, A3s, A3d)
    ex3 = edge1(*heads_of(asrc3), *heads_of(adst3), src, dst)
    acc3, den3 = _sc_aggregate3(h3, ex3, src2d, dst2d)

    den3r = den3.reshape(2, NPAD)
    h3f = _tc_final_feat(acc3[0, :N], acc3[1, :N], den3r[0, :N, None],
                         den3r[1, :N, None],
                         b3.reshape(1, -1), g3.reshape(1, -1),
                         be3.reshape(1, -1))

    h3p = jnp.concatenate([h3f, jnp.zeros((NP4 - N, 16), jnp.float32)], axis=0)
    batchp = jnp.concatenate(
        [batch.astype(jnp.int32), jnp.full((NP4 - N,), G, jnp.int32)])
    mx, sm, ct = _sc_pool(h3p, batchp)
    out = _tc_head(mx[0], mx[1], sm[0], sm[1], ct[0], ct[1], u,
                   Wc1, bc1.reshape(1, -1), Wc2, bc2.reshape(1, -1),
                   Wc3, bc3.reshape(1, -1))
    return out[:, 0]


# speedup vs baseline: 101.9757x; 1.0367x over previous
import functools

import jax
import jax.numpy as jnp
from jax import lax
from jax.experimental import pallas as pl
from jax.experimental.pallas import tpu as pltpu
from jax.experimental.pallas import tpu_sc as plsc

N = 50000
E = 800000
G = 64
F_IN = 128
HID = 16
HEADS = 4
N_GLOBAL = 10

EE = E + N
NW = 32
W1E = 2048
W2E = 2048
EP = 851968
NPAD = 50048
ROWS_PER_SUB = NPAD // 16
NP4 = 50176
POOL_CHUNK = NP4 // NW
POOL_WIN = 224
GT = 128

_MESH = plsc.VectorSubcoreMesh(core_axis_name="c", subcore_axis_name="s")


def _f32(shape):
    return jax.ShapeDtypeStruct(shape, jnp.float32)


def _make_edge_logits(H):
    span = EP // NW
    n_win = span // W1E
    QG = W1E // 128

    def body(*refs):
        a_hbm = refs[:H]
        b_hbm = refs[H:2 * H]
        src, dst, ex_out = refs[2 * H:2 * H + 3]
        rest = refs[2 * H + 3:]
        atabs = rest[:H]
        btabs = rest[H:2 * H]
        srcw, dstw, ea, eb, exb, tbuf, sem, sem2 = rest[2 * H:]
        c = lax.axis_index("c")
        s = lax.axis_index("s")
        wid = s * 2 + c
        base = wid * span

        for k in range(H):
            @pl.when(s == k)
            def _stage_a(k=k):
                pltpu.sync_copy(a_hbm[k], tbuf)
                pltpu.sync_copy(tbuf, atabs[k])

            @pl.when(s == H + k)
            def _stage_b(k=k):
                pltpu.sync_copy(b_hbm[k], tbuf)
                pltpu.sync_copy(tbuf, btabs[k])

        plsc.subcore_barrier()

        def win(i, _):
            off = pl.multiple_of(base + i * W1E, 128)
            pltpu.sync_copy(src.at[pl.ds(off, W1E)], srcw)
            pltpu.sync_copy(dst.at[pl.ds(off, W1E)], dstw)
            for head in range(H):
                cps = []
                for q in range(QG):
                    cps.append(pltpu.async_copy(
                        atabs[head].at[srcw.at[pl.ds(q * 128, 128)]],
                        ea.at[pl.ds(q * 128, 128)], sem))
                    cps.append(pltpu.async_copy(
                        btabs[head].at[dstw.at[pl.ds(q * 128, 128)]],
                        eb.at[pl.ds(q * 128, 128)], sem2))
                for cp in cps:
                    cp.wait()

                def grp(j, _):
                    v = ea[pl.ds(j * 16, 16)] + eb[pl.ds(j * 16, 16)]
                    v = jnp.where(v >= 0.0, v, v * jnp.float32(0.2))
                    v = jnp.exp(v)
                    eid = lax.iota(jnp.int32, 16) + (off + j * 16)
                    v = jnp.where(eid < EE, v, jnp.float32(0.0))
                    exb[pl.ds(j * 16, 16)] = v
                    return 0

                lax.fori_loop(0, W1E // 16, grp, 0)
                pltpu.sync_copy(
                    exb, ex_out.at[pl.ds(pl.multiple_of(head * EP + off, 128),
                                         W1E)])
            return 0

        lax.fori_loop(0, n_win, win, 0)

    return pl.kernel(
        body,
        out_type=_f32((H * EP,)),
        mesh=_MESH,
        compiler_params=pltpu.CompilerParams(use_tc_tiling_on_sc=False),
        scratch_types=(
            [pltpu.VMEM_SHARED((N,), jnp.float32) for _ in range(2 * H)]
            + [
                pltpu.VMEM((W1E,), jnp.int32),
                pltpu.VMEM((W1E,), jnp.int32),
                pltpu.VMEM((W1E,), jnp.float32),
                pltpu.VMEM((W1E,), jnp.float32),
                pltpu.VMEM((W1E,), jnp.float32),
                pltpu.VMEM((N,), jnp.float32),
                pltpu.SemaphoreType.DMA,
                pltpu.SemaphoreType.DMA,
            ]
        ),
    )


def _aggregate_body(hcat, ex, src2d, dst2d, acc_out, den_out,
                    acc_s, den_s, srcw, dstw, didx0, didx1, ex0w, ex1w,
                    msg0, msg1, msg2, semg, sems, semd):
    c = lax.axis_index("c")
    s = lax.axis_index("s")
    cN = c * N
    h0 = 2 * c
    h1 = 2 * c + 1
    r0 = s * ROWS_PER_SUB
    msgs = (msg0, msg1, msg2)

    def zf(i, _):
        msg0[i, pl.ds(0, 16)] = jnp.zeros((16,), jnp.float32)
        msg0[i, pl.ds(16, 16)] = jnp.zeros((16,), jnp.float32)
        return 0

    lax.fori_loop(0, 128, zf, 0)

    def zfd(i, _):
        ex0w[pl.ds(i * 16, 16)] = jnp.zeros((16,), jnp.float32)
        return 0

    lax.fori_loop(0, W2E // 16, zfd, 0)
    for k in range(24):
        pltpu.sync_copy(msg0, acc_s.at[pl.ds(r0 + k * 128, 128)])
    pltpu.sync_copy(msg0.at[pl.ds(0, 56)], acc_s.at[pl.ds(r0 + 3072, 56)])
    for k in range(3):
        pltpu.sync_copy(ex0w, den_s.at[pl.ds(r0 * 2 + k * 2048, 2048)])
    pltpu.sync_copy(ex0w.at[pl.ds(0, 112)],
                    den_s.at[pl.ds(r0 * 2 + 6144, 112)])
    plsc.subcore_barrier()

    span = EP // 16
    base = s * span
    n_win = span // W2E

    def win(i, _):
        off = pl.multiple_of(base + i * W2E, 128)
        row0 = pl.multiple_of(off // 128, 8)
        pltpu.sync_copy(src2d.at[pl.ds(row0, 16)], srcw)
        pltpu.sync_copy(dst2d.at[pl.ds(row0, 16)], dstw)
        pltpu.sync_copy(ex.at[pl.ds(pl.multiple_of(h0 * EP + off, 128),
                                     W2E)], ex0w)
        pltpu.sync_copy(ex.at[pl.ds(pl.multiple_of(h1 * EP + off, 128),
                                     W2E)], ex1w)
        for q in range(16):
            for j in range(8):
                sl = pl.ds(j * 16, 16)
                srcw[q, sl] = srcw[q, sl] + cN
                dv = dstw[q, sl]
                didx0[q, sl] = dv * 2
                didx1[q, sl] = dv * 2 + 1

        gath = {0: pltpu.async_copy(hcat.at[srcw.at[0]], msgs[0], semg)}
        scat = {}
        for q in range(16):
            if q >= 2:
                for cp in scat[q - 2]:
                    cp.wait()
            if q + 1 < 16:
                gath[q + 1] = pltpu.async_copy(
                    hcat.at[srcw.at[q + 1]], msgs[(q + 1) % 3], semg)
            gath[q].wait()
            buf = msgs[q % 3]
            for g in range(8):
                x0v = ex0w[pl.ds(q * 128 + g * 16, 16)]
                x1v = ex1w[pl.ds(q * 128 + g * 16, 16)]
                for j in range(16):
                    e = g * 16 + j
                    buf[e, pl.ds(0, 16)] = buf[e, pl.ds(0, 16)] * x0v[j]
                    buf[e, pl.ds(16, 16)] = buf[e, pl.ds(16, 16)] * x1v[j]
            scat[q] = [
                pltpu.async_copy(buf, acc_s.at[dstw.at[q]], sems, add=True),
                pltpu.async_copy(ex0w.at[pl.ds(q * 128, 128)],
                                 den_s.at[didx0.at[q]], semd, add=True),
                pltpu.async_copy(ex1w.at[pl.ds(q * 128, 128)],
                                 den_s.at[didx1.at[q]], semd, add=True),
            ]
        for q in (14, 15):
            for cp in scat[q]:
                cp.wait()
        return 0

    lax.fori_loop(0, n_win, win, 0)
    plsc.subcore_barrier()
    for k in range(24):
        pltpu.sync_copy(acc_s.at[pl.ds(r0 + k * 128, 128)], msg0)
        pltpu.sync_copy(msg0, acc_out.at[c, pl.ds(r0 + k * 128, 128)])
    pltpu.sync_copy(acc_s.at[pl.ds(r0 + 3072, 56)], msg0.at[pl.ds(0, 56)])
    pltpu.sync_copy(msg0.at[pl.ds(0, 56)],
                    acc_out.at[c, pl.ds(r0 + 3072, 56)])
    dbase = c * (NPAD * 2) + r0 * 2
    for k in range(3):
        pltpu.sync_copy(den_s.at[pl.ds(r0 * 2 + k * 2048, 2048)], ex0w)
        pltpu.sync_copy(ex0w, den_out.at[pl.ds(
            pl.multiple_of(dbase + k * 2048, 8), 2048)])
    pltpu.sync_copy(den_s.at[pl.ds(r0 * 2 + 6144, 112)],
                    ex0w.at[pl.ds(0, 112)])
    pltpu.sync_copy(ex0w.at[pl.ds(0, 112)],
                    den_out.at[pl.ds(pl.multiple_of(dbase + 6144, 8), 112)])


_sc_aggregate = pl.kernel(
    _aggregate_body,
    out_type=(_f32((2, NPAD, 32)), _f32((2 * NPAD * 2,))),
    mesh=_MESH,
    compiler_params=pltpu.CompilerParams(use_tc_tiling_on_sc=False),
    scratch_types=[
        pltpu.VMEM_SHARED((NPAD, 32), jnp.float32),
        pltpu.VMEM_SHARED((NPAD * 2,), jnp.float32),
        pltpu.VMEM((16, 128), jnp.int32),
        pltpu.VMEM((16, 128), jnp.int32),
        pltpu.VMEM((16, 128), jnp.int32),
        pltpu.VMEM((16, 128), jnp.int32),
        pltpu.VMEM((W2E,), jnp.float32),
        pltpu.VMEM((W2E,), jnp.float32),
        pltpu.VMEM((128, 32), jnp.float32),
        pltpu.VMEM((128, 32), jnp.float32),
        pltpu.VMEM((128, 32), jnp.float32),
        pltpu.SemaphoreType.DMA,
        pltpu.SemaphoreType.DMA,
        pltpu.SemaphoreType.DMA,
    ],
)


def _aggregate3_body(h3, ex, src2d, dst2d, acc_out, den_out,
                     acc_s, den_s, srcw, dstw, ex0w, msg0, msg1, msg2,
                     semg, sems, semd):
    c = lax.axis_index("c")
    s = lax.axis_index("s")
    wid = s * 2 + c
    r0 = s * ROWS_PER_SUB
    msgs = (msg0, msg1, msg2)

    def zf(i, _):
        msg0[i, pl.ds(0, 16)] = jnp.zeros((16,), jnp.float32)
        return 0

    lax.fori_loop(0, 128, zf, 0)

    def zfd(i, _):
        ex0w[pl.ds(i * 16, 16)] = jnp.zeros((16,), jnp.float32)
        return 0

    lax.fori_loop(0, W2E // 16, zfd, 0)
    for k in range(24):
        pltpu.sync_copy(msg0, acc_s.at[pl.ds(r0 + k * 128, 128)])
    pltpu.sync_copy(msg0.at[pl.ds(0, 56)], acc_s.at[pl.ds(r0 + 3072, 56)])
    pltpu.sync_copy(ex0w, den_s.at[pl.ds(r0, 2048)])
    pltpu.sync_copy(ex0w.at[pl.ds(0, 1080)], den_s.at[pl.ds(r0 + 2048, 1080)])
    plsc.subcore_barrier()

    span = EP // NW
    base = wid * span
    n_win = span // W2E

    def win(i, _):
        off = pl.multiple_of(base + i * W2E, 128)
        row0 = pl.multiple_of(off // 128, 8)
        pltpu.sync_copy(src2d.at[pl.ds(row0, 16)], srcw)
        pltpu.sync_copy(dst2d.at[pl.ds(row0, 16)], dstw)
        pltpu.sync_copy(ex.at[pl.ds(off, W2E)], ex0w)

        gath = {0: pltpu.async_copy(h3.at[srcw.at[0]], msgs[0], semg)}
        scat = {}
        for q in range(16):
            if q >= 2:
                for cp in scat[q - 2]:
                    cp.wait()
            if q + 1 < 16:
                gath[q + 1] = pltpu.async_copy(
                    h3.at[srcw.at[q + 1]], msgs[(q + 1) % 3], semg)
            gath[q].wait()
            buf = msgs[q % 3]
            for g in range(8):
                x0v = ex0w[pl.ds(q * 128 + g * 16, 16)]
                for j in range(16):
                    e = g * 16 + j
                    buf[e, pl.ds(0, 16)] = buf[e, pl.ds(0, 16)] * x0v[j]
            scat[q] = [
                pltpu.async_copy(buf, acc_s.at[dstw.at[q]], sems, add=True),
                pltpu.async_copy(ex0w.at[pl.ds(q * 128, 128)],
                                 den_s.at[dstw.at[q]], semd, add=True),
            ]
        for q in (14, 15):
            for cp in scat[q]:
                cp.wait()
        return 0

    lax.fori_loop(0, n_win, win, 0)
    plsc.subcore_barrier()
    for k in range(24):
        pltpu.sync_copy(acc_s.at[pl.ds(r0 + k * 128, 128)], msg0)
        pltpu.sync_copy(msg0, acc_out.at[c, pl.ds(r0 + k * 128, 128)])
    pltpu.sync_copy(acc_s.at[pl.ds(r0 + 3072, 56)], msg0.at[pl.ds(0, 56)])
    pltpu.sync_copy(msg0.at[pl.ds(0, 56)],
                    acc_out.at[c, pl.ds(r0 + 3072, 56)])
    dbase = c * NPAD + r0
    pltpu.sync_copy(den_s.at[pl.ds(r0, 2048)], ex0w)
    pltpu.sync_copy(ex0w, den_out.at[pl.ds(pl.multiple_of(dbase, 8), 2048)])
    pltpu.sync_copy(den_s.at[pl.ds(r0 + 2048, 1080)],
                    ex0w.at[pl.ds(0, 1080)])
    pltpu.sync_copy(ex0w.at[pl.ds(0, 1080)],
                    den_out.at[pl.ds(pl.multiple_of(dbase + 2048, 8), 1080)])


_sc_aggregate3 = pl.kernel(
    _aggregate3_body,
    out_type=(_f32((2, NPAD, 16)), _f32((2 * NPAD,))),
    mesh=_MESH,
    compiler_params=pltpu.CompilerParams(use_tc_tiling_on_sc=False),
    scratch_types=[
        pltpu.VMEM_SHARED((NPAD, 16), jnp.float32),
        pltpu.VMEM_SHARED((NPAD,), jnp.float32),
        pltpu.VMEM((16, 128), jnp.int32),
        pltpu.VMEM((16, 128), jnp.int32),
        pltpu.VMEM((W2E,), jnp.float32),
        pltpu.VMEM((128, 16), jnp.float32),
        pltpu.VMEM((128, 16), jnp.float32),
        pltpu.VMEM((128, 16), jnp.float32),
        pltpu.SemaphoreType.DMA,
        pltpu.SemaphoreType.DMA,
        pltpu.SemaphoreType.DMA,
    ],
)


def _pool_body(h, batch, maxo, sumo, cnto,
               maxt, sumt, cntt, spmax, spsum, spcnt, hwin, bwin, red, res):
    c = lax.axis_index("c")
    s = lax.axis_index("s")
    wid = s * 2 + c

    def init(r, _):
        maxt[r, pl.ds(0, 16)] = jnp.full((16,), -3e38, jnp.float32)
        sumt[r, pl.ds(0, 16)] = jnp.zeros((16,), jnp.float32)
        cntt[r, pl.ds(0, 16)] = jnp.zeros((16,), jnp.float32)
        return 0

    lax.fori_loop(0, GT, init, 0)

    base = wid * POOL_CHUNK

    def win(i, _):
        off = base + i * POOL_WIN
        pltpu.sync_copy(h.at[pl.ds(off, POOL_WIN)], hwin)
        pltpu.sync_copy(batch.at[pl.ds(off, POOL_WIN)], bwin)

        def row(i, _):
            r0 = i * 16
            bv = bwin[pl.ds(r0, 16)]
            for j in range(16):
                b = bv[j]
                hv = hwin[r0 + j, pl.ds(0, 16)]
                maxt[b, pl.ds(0, 16)] = jnp.maximum(maxt[b, pl.ds(0, 16)], hv)
                sumt[b, pl.ds(0, 16)] = sumt[b, pl.ds(0, 16)] + hv
                cntt[b, pl.ds(0, 16)] = cntt[b, pl.ds(0, 16)] + jnp.float32(1.0)
            return 0

        lax.fori_loop(0, POOL_WIN // 16, row, 0)
        return 0

    lax.fori_loop(0, POOL_CHUNK // POOL_WIN, win, 0)

    pltpu.sync_copy(maxt, spmax.at[s])
    pltpu.sync_copy(sumt, spsum.at[s])
    pltpu.sync_copy(cntt, spcnt.at[s])
    plsc.subcore_barrier()

    rr = GT // 16
    for tab, out in ((spmax, maxo), (spsum, sumo), (spcnt, cnto)):
        pltpu.sync_copy(tab.at[:, pl.ds(s * rr, rr)], red)
        is_max = tab is spmax
        for r5 in range(rr):
            m = red[0, r5, pl.ds(0, 16)]
            for t in range(1, 16):
                v = red[t, r5, pl.ds(0, 16)]
                m = jnp.maximum(m, v) if is_max else m + v
            res[r5, pl.ds(0, 16)] = m
        pltpu.sync_copy(res, out.at[c, pl.ds(s * rr, rr)])


_sc_pool = pl.kernel(
    _pool_body,
    out_type=(_f32((2, GT, 16)), _f32((2, GT, 16)), _f32((2, GT, 16))),
    mesh=_MESH,
    compiler_params=pltpu.CompilerParams(use_tc_tiling_on_sc=False),
    scratch_types=[
        pltpu.VMEM((GT, 16), jnp.float32),
        pltpu.VMEM((GT, 16), jnp.float32),
        pltpu.VMEM((GT, 16), jnp.float32),
        pltpu.VMEM_SHARED((16, GT, 16), jnp.float32),
        pltpu.VMEM_SHARED((16, GT, 16), jnp.float32),
        pltpu.VMEM_SHARED((16, GT, 16), jnp.float32),
        pltpu.VMEM((POOL_WIN, 16), jnp.float32),
        pltpu.VMEM((POOL_WIN,), jnp.int32),
        pltpu.VMEM((16, GT // 16, 16), jnp.float32),
        pltpu.VMEM((GT // 16, 16), jnp.float32),
    ],
)


_BR = 1000


def _proj_kernel(x_ref, w_ref, as_ref, ad_ref, h_ref, s_ref, d_ref):
    h = jnp.dot(x_ref[...], w_ref[...], preferred_element_type=jnp.float32)
    h_ref[...] = h
    s_ref[...] = jnp.dot(h, as_ref[...], preferred_element_type=jnp.float32)
    d_ref[...] = jnp.dot(h, ad_ref[...], preferred_element_type=jnp.float32)


def _tc_project(x, Wm, As, Ad):
    k = x.shape[1]
    co = Wm.shape[1]
    ch = As.shape[1]
    return pl.pallas_call(
        _proj_kernel,
        grid=(N // _BR,),
        in_specs=[
            pl.BlockSpec((_BR, k), lambda i: (i, 0)),
            pl.BlockSpec((k, co), lambda i: (0, 0)),
            pl.BlockSpec((co, ch), lambda i: (0, 0)),
            pl.BlockSpec((co, ch), lambda i: (0, 0)),
        ],
        out_specs=[
            pl.BlockSpec((_BR, co), lambda i: (i, 0)),
            pl.BlockSpec((_BR, ch), lambda i: (i, 0)),
            pl.BlockSpec((_BR, ch), lambda i: (i, 0)),
        ],
        out_shape=[_f32((N, co)), _f32((N, ch)), _f32((N, ch))],
    )(x, Wm, As, Ad)


import math

_BN_INV = float(1.0 / math.sqrt(1.0 + 1e-5))


def _elu(m):
    neg = jnp.where(m > 0.0, 0.0, m)
    return jnp.where(m > 0.0, m, jnp.exp(neg) - 1.0)


def _comb_kernel(aA_ref, aB_ref, d0_ref, d1_ref, r_ref, b_ref, g_ref,
                 be_ref, w_ref, as_ref, ad_ref, h_ref, s_ref, d_ref):
    m = jnp.concatenate([aA_ref[...], aB_ref[...]], axis=1)
    den4 = jnp.concatenate([d0_ref[...], d1_ref[...]], axis=1)
    denr = jnp.dot(den4, r_ref[...], preferred_element_type=jnp.float32)
    m = m / (denr + 1e-16)
    m = m + b_ref[...]
    m = m * _BN_INV * g_ref[...] + be_ref[...]
    m = _elu(m)
    h = jnp.dot(m, w_ref[...], preferred_element_type=jnp.float32)
    h_ref[...] = h
    s_ref[...] = jnp.dot(h, as_ref[...], preferred_element_type=jnp.float32)
    d_ref[...] = jnp.dot(h, ad_ref[...], preferred_element_type=jnp.float32)


def _tc_combine_project(accA, accB, d0, d1, R, b, g, be, Wm, As, Ad):
    co = Wm.shape[1]
    ch = As.shape[1]
    return pl.pallas_call(
        _comb_kernel,
        grid=(N // _BR,),
        in_specs=[
            pl.BlockSpec((_BR, 32), lambda i: (i, 0)),
            pl.BlockSpec((_BR, 32), lambda i: (i, 0)),
            pl.BlockSpec((_BR, 2), lambda i: (i, 0)),
            pl.BlockSpec((_BR, 2), lambda i: (i, 0)),
            pl.BlockSpec((4, 64), lambda i: (0, 0)),
            pl.BlockSpec((1, 64), lambda i: (0, 0)),
            pl.BlockSpec((1, 64), lambda i: (0, 0)),
            pl.BlockSpec((1, 64), lambda i: (0, 0)),
            pl.BlockSpec((64, co), lambda i: (0, 0)),
            pl.BlockSpec((co, ch), lambda i: (0, 0)),
            pl.BlockSpec((co, ch), lambda i: (0, 0)),
        ],
        out_specs=[
            pl.BlockSpec((_BR, co), lambda i: (i, 0)),
            pl.BlockSpec((_BR, ch), lambda i: (i, 0)),
            pl.BlockSpec((_BR, ch), lambda i: (i, 0)),
        ],
        out_shape=[_f32((N, co)), _f32((N, ch)), _f32((N, ch))],
    )(accA, accB, d0, d1, R, b, g, be, Wm, As, Ad)


def _final_feat_kernel(a0_ref, a1_ref, n0_ref, n1_ref, b_ref, g_ref, be_ref,
                       o_ref):
    m = (a0_ref[...] + a1_ref[...]) / (n0_ref[...] + n1_ref[...] + 1e-16)
    m = m + b_ref[...]
    m = m * _BN_INV * g_ref[...] + be_ref[...]
    o_ref[...] = _elu(m)


def _tc_final_feat(a0, a1, n0, n1, b, g, be):
    return pl.pallas_call(
        _final_feat_kernel,
        grid=(N // _BR,),
        in_specs=[
            pl.BlockSpec((_BR, 16), lambda i: (i, 0)),
            pl.BlockSpec((_BR, 16), lambda i: (i, 0)),
            pl.BlockSpec((_BR, 1), lambda i: (i, 0)),
            pl.BlockSpec((_BR, 1), lambda i: (i, 0)),
            pl.BlockSpec((1, 16), lambda i: (0, 0)),
            pl.BlockSpec((1, 16), lambda i: (0, 0)),
            pl.BlockSpec((1, 16), lambda i: (0, 0)),
        ],
        out_specs=[pl.BlockSpec((_BR, 16), lambda i: (i, 0))],
        out_shape=[_f32((N, 16))],
    )(a0, a1, n0, n1, b, g, be)[0]


def _head_kernel(mx0_ref, mx1_ref, sm0_ref, sm1_ref, ct0_ref, ct1_ref, u_ref,
                 w1_ref, b1_ref, w2_ref, b2_ref, w3_ref, b3_ref, o_ref):
    mx = jnp.maximum(mx0_ref[...], mx1_ref[...])[:G]
    sm = (sm0_ref[...] + sm1_ref[...])[:G]
    ct = (ct0_ref[...] + ct1_ref[...])[:G, 0:1]
    mean = sm / jnp.maximum(ct, 1.0)
    xmax = jnp.where(ct > 0.0, mx, 0.0)
    z = jnp.concatenate([mean, xmax, u_ref[...]], axis=1)
    z = jnp.maximum(jnp.dot(z, w1_ref[...], preferred_element_type=jnp.float32)
                    + b1_ref[...], 0.0)
    z = jnp.maximum(jnp.dot(z, w2_ref[...], preferred_element_type=jnp.float32)
                    + b2_ref[...], 0.0)
    z = jnp.dot(z, w3_ref[...], preferred_element_type=jnp.float32) + b3_ref[...]
    o_ref[...] = 1.0 / (1.0 + jnp.exp(-z))


def _tc_head(mx0, mx1, sm0, sm1, ct0, ct1, u, w1, b1, w2, b2, w3, b3):
    return pl.pallas_call(
        _head_kernel,
        out_shape=_f32((G, 1)),
    )(mx0, mx1, sm0, sm1, ct0, ct1, u, w1, b1, w2, b2, w3, b3)


def _attn_mat(a):
    heads, ch = a.shape
    m = jnp.zeros((heads * ch, heads), jnp.float32)
    for h in range(heads):
        m = m.at[h * ch:(h + 1) * ch, h].set(a[h])
    return m


def kernel(x, u, W1, a_src1, a_dst1, b1, W2, a_src2, a_dst2, b2, W3, a_src3,
           a_dst3, b3, g1, be1, g2, be2, g3, be3, Wc1, bc1, Wc2, bc2, Wc3,
           bc3, edge_index, batch):
    loops = jnp.arange(N, dtype=edge_index.dtype)
    pad = (jnp.arange(EP - EE, dtype=edge_index.dtype) * 97) % N
    src = jnp.concatenate([edge_index[0], loops, pad])
    dst = jnp.concatenate([edge_index[1], loops, pad])
    src2d = src.reshape(EP // 128, 128)
    dst2d = dst.reshape(EP // 128, 128)

    A1s, A1d = _attn_mat(a_src1), _attn_mat(a_dst1)
    A2s, A2d = _attn_mat(a_src2), _attn_mat(a_dst2)
    A3s, A3d = _attn_mat(a_src3), _attn_mat(a_dst3)
    R = jnp.repeat(jnp.eye(4, dtype=jnp.float32), 16, axis=1)

    edge4 = _make_edge_logits(4)
    edge1 = _make_edge_logits(1)

    def heads_of(a):
        return [a[:, k].copy() for k in range(a.shape[1])]

    h, asrc, adst = _tc_project(x, W1, A1s, A1d)
    ex = edge4(*heads_of(asrc), *heads_of(adst), src, dst)
    hcat = jnp.concatenate([h[:, :32], h[:, 32:]], axis=0)
    acc, den = _sc_aggregate(hcat, ex, src2d, dst2d)

    den2 = den.reshape(2, NPAD, 2)
    h, asrc, adst = _tc_combine_project(
        acc[0, :N], acc[1, :N], den2[0, :N], den2[1, :N], R,
        b1.reshape(1, -1), g1.reshape(1, -1), be1.reshape(1, -1),
        W2, A2s, A2d)
    ex = edge4(*heads_of(asrc), *heads_of(adst), src, dst)
    hcat = jnp.concatenate([h[:, :32], h[:, 32:]], axis=0)
    acc, den = _sc_aggregate(hcat, ex, src2d, dst2d)

    den2 = den.reshape(2, NPAD, 2)
    h3, asrc3, adst3 = _tc_combine_project(
        acc[0, :N], acc[1, :N], den2[0, :N], den2[1, :N], R,
        b2.reshape(1, -1), g2.reshape(1, -1), be2.reshape(1, -1),
        W3, A3s, A3d)
    ex3 = edge1(*heads_of(asrc3), *heads_of(adst3), src, dst)
    acc3, den3 = _sc_aggregate3(h3, ex3, src2d, dst2d)

    den3r = den3.reshape(2, NPAD)
    h3f = _tc_final_feat(acc3[0, :N], acc3[1, :N], den3r[0, :N, None],
                         den3r[1, :N, None],
                         b3.reshape(1, -1), g3.reshape(1, -1),
                         be3.reshape(1, -1))

    h3p = jnp.concatenate([h3f, jnp.zeros((NP4 - N, 16), jnp.float32)], axis=0)
    batchp = jnp.concatenate(
        [batch.astype(jnp.int32), jnp.full((NP4 - N,), G, jnp.int32)])
    mx, sm, ct = _sc_pool(h3p, batchp)
    out = _tc_head(mx[0], mx[1], sm[0], sm[1], ct[0], ct[1], u,
                   Wc1, bc1.reshape(1, -1), Wc2, bc2.reshape(1, -1),
                   Wc3, bc3.reshape(1, -1))
    return out[:, 0]
